# baseline (speedup 1.0000x reference)
import functools

import jax
import jax.numpy as jnp
from jax import lax
from jax.experimental import pallas as pl
from jax.experimental.pallas import tpu as pltpu
from jax.experimental.pallas import tpu_sc as plsc

BETA = 0.1
NBINS = 256
LANES = 16
NW = 32
CH = 24576
STAT_BLK = 786432


def _stats_body(x_ref, min_ref, max_ref, sab_ref):
    i = pl.program_id(0)
    x = x_ref[...]
    bmin = jnp.min(x, keepdims=True)
    bmax = jnp.max(x, keepdims=True)
    bsum = jnp.sum(jnp.abs(x), keepdims=True)

    @pl.when(i == 0)
    def _():
        min_ref[...] = bmin
        max_ref[...] = bmax
        sab_ref[...] = bsum

    @pl.when(i != 0)
    def _():
        min_ref[...] = jnp.minimum(min_ref[...], bmin)
        max_ref[...] = jnp.maximum(max_ref[...], bmax)
        sab_ref[...] = sab_ref[...] + bsum


def _stats(x2, grid=8):
    rows, width = x2.shape
    blk = rows // grid
    out = pl.pallas_call(
        _stats_body,
        grid=(grid,),
        in_specs=[pl.BlockSpec((blk, width), lambda i: (i, 0))],
        out_specs=[pl.BlockSpec((1, 1), lambda i: (0, 0))] * 3,
        out_shape=[jax.ShapeDtypeStruct((1, 1), jnp.float32)] * 3,
    )(x2)
    return out[0][0, 0], out[1][0, 0], out[2][0, 0]


def _sc_stats(flat):
    per = int(flat.shape[0]) // NW
    nch = per // CH
    assert nch % 2 == 0
    mesh = plsc.VectorSubcoreMesh(core_axis_name="c", subcore_axis_name="s")

    def body(src, out, buf, st, sem0, sem1):
        cid = lax.axis_index("c")
        sid = lax.axis_index("s")
        wid = sid * 2 + cid
        base = wid * per
        sems = (sem0, sem1)

        pltpu.async_copy(src.at[pl.ds(base, CH)], buf.at[0], sem0)
        pltpu.async_copy(src.at[pl.ds(base + CH, CH)], buf.at[1], sem1)

        big = jnp.full((LANES,), 3.4028234e38, jnp.float32)
        zero = jnp.zeros((LANES,), jnp.float32)
        init = ((big,) * 8, ((-big),) * 8, (zero,) * 8)

        def _chunks(ci, carry):
            for b in (0, 1):
                pltpu.make_async_copy(
                    src.at[pl.ds(0, CH)], buf.at[b], sems[b]).wait()

                def _vecs(i, c):
                    mns, mxs, sas = c
                    mns, mxs, sas = list(mns), list(mxs), list(sas)
                    for u in range(8):
                        v = buf[b, pl.ds((i + u) * LANES, LANES)]
                        mns[u] = jnp.minimum(mns[u], v)
                        mxs[u] = jnp.maximum(mxs[u], v)
                        sas[u] = sas[u] + jnp.abs(v)
                    return (tuple(mns), tuple(mxs), tuple(sas))

                carry = pl.loop(0, CH // LANES, step=8,
                                init_carry=carry)(_vecs)

                nxt = ci + b + 2

                @pl.when(nxt < nch)
                def _():
                    pltpu.async_copy(src.at[pl.ds(base + nxt * CH, CH)],
                                     buf.at[b], sems[b])
            return carry

        mns, mxs, sas = pl.loop(0, nch, step=2, init_carry=init)(_chunks)
        mn, mx, sa = mns[0], mxs[0], sas[0]
        for u in range(1, 8):
            mn = jnp.minimum(mn, mns[u])
            mx = jnp.maximum(mx, mxs[u])
            sa = sa + sas[u]
        st[0, :] = mn
        st[1, :] = mx
        st[2, :] = sa
        pltpu.sync_copy(st, out.at[wid])

    kfn = pl.kernel(
        body,
        out_type=jax.ShapeDtypeStruct((NW, 3, LANES), jnp.float32),
        mesh=mesh,
        scratch_types=[
            pltpu.VMEM((2, CH), jnp.float32),
            pltpu.VMEM((3, LANES), jnp.float32),
            pltpu.SemaphoreType.DMA,
            pltpu.SemaphoreType.DMA,
        ],
        compiler_params=pltpu.CompilerParams(
            use_tc_tiling_on_sc=False, needs_layout_passes=False),
    )
    return kfn(flat)


def _histograms(flat_list, par):
    nt = len(flat_list)
    per_tiles = tuple(int(f.shape[0]) // NW for f in flat_list)
    mesh = plsc.VectorSubcoreMesh(core_axis_name="c", subcore_axis_name="s")

    def body(*refs):
        srcs = refs[:nt]
        par_ref = refs[nt]
        out = refs[nt + 1]
        buf = refs[nt + 2]
        hists = refs[nt + 3:nt + 3 + nt]
        pv = refs[nt + 3 + nt]
        sem0, sem1 = refs[nt + 4 + nt:nt + 6 + nt]

        cid = lax.axis_index("c")
        sid = lax.axis_index("s")
        wid = sid * 2 + cid

        zero16 = jnp.zeros((LANES,), jnp.float32)

        @pl.loop(0, NBINS)
        def _zero(r):
            for h in hists:
                h[r, :] = zero16

        pltpu.sync_copy(par_ref, pv)

        lanes = lax.iota(jnp.int32, LANES)
        ones = jnp.ones((LANES,), jnp.float32)
        sems = (sem0, sem1)

        for t in range(nt):
            src = srcs[t]
            hist = hists[t]
            per = per_tiles[t]
            nch = per // CH
            base = wid * per
            minv = pv[t, :]
            scv = pv[nt + t, :]

            pltpu.async_copy(src.at[pl.ds(base, CH)], buf.at[0], sem0)
            pltpu.async_copy(src.at[pl.ds(base + CH, CH)], buf.at[1], sem1)

            @pl.loop(0, nch, step=2)
            def _chunks(ci):
                for b in (0, 1):
                    cur = ci + b

                    @pl.when(cur < nch)
                    def _():
                        pltpu.make_async_copy(
                            src.at[pl.ds(0, CH)], buf.at[b], sems[b]).wait()

                        @plsc.parallel_loop(0, CH // LANES, unroll=8)
                        def _vecs(i):
                            v = buf[b, pl.ds(i * LANES, LANES)]
                            xf = (v - minv) * scv
                            bi = xf.astype(jnp.int32)
                            bi = jnp.minimum(bi, NBINS - 1)
                            plsc.addupdate_scatter(hist, [bi, lanes], ones)

                        nxt = cur + 2

                        @pl.when(nxt < nch)
                        def _():
                            pltpu.async_copy(
                                src.at[pl.ds(base + nxt * CH, CH)],
                                buf.at[b], sems[b])

        for t in range(nt):
            pltpu.sync_copy(hists[t], out.at[t, wid])

    kfn = pl.kernel(
        body,
        out_type=jax.ShapeDtypeStruct((nt, NW, NBINS, LANES), jnp.float32),
        mesh=mesh,
        scratch_types=(
            [pltpu.VMEM((2, CH), jnp.float32)]
            + [pltpu.VMEM((NBINS, LANES), jnp.float32)] * nt
            + [pltpu.VMEM((2 * nt, LANES), jnp.float32),
               pltpu.SemaphoreType.DMA,
               pltpu.SemaphoreType.DMA]),
        compiler_params=pltpu.CompilerParams(
            use_tc_tiling_on_sc=False, needs_layout_passes=False),
    )
    return kfn(*flat_list, par)


def _finish_body(ns, hc_ref, hab_ref, s_ref, o_ref):
    ent = jnp.float32(0.0)
    spars = jnp.float32(0.0)
    for t in range(3):
        n = jnp.float32(ns[t])
        href, k = (hab_ref, t) if t < 2 else (hc_ref, 0)
        h = jnp.sum(href[k], axis=1, keepdims=True)
        p = h / n
        ent = ent + (-jnp.sum(p * jnp.log2(p + 1e-08)))
        spars = spars + s_ref[0, t] / n
    o_ref[...] = jnp.reshape(spars / 3.0 + BETA * (ent / 3.0), (1, 1))


def _finish(hists_c, hists_ab, sums, ns):
    hc = hists_c.transpose(0, 2, 1, 3).reshape(1, NBINS, NW * LANES)
    hab = hists_ab.transpose(0, 2, 1, 3).reshape(2, NBINS, NW * LANES)
    out = pl.pallas_call(
        functools.partial(_finish_body, ns),
        out_shape=jax.ShapeDtypeStruct((1, 1), jnp.float32),
    )(hc, hab, sums.reshape(1, 3))
    return out[0, 0]


def _par(mins, maxs):
    scale = 256.0 / (maxs - mins + 1e-08)
    k = mins.shape[0]
    return jnp.concatenate(
        [jnp.broadcast_to(mins[:, None], (k, LANES)),
         jnp.broadcast_to(scale[:, None], (k, LANES))], axis=0)


def kernel(feat_a, feat_b, feat_c):
    flat_c = feat_c.reshape(-1)
    pc = _sc_stats(flat_c)
    mn_c = jnp.min(pc[:, 0, :])
    mx_c = jnp.max(pc[:, 1, :])
    sa_c = jnp.sum(pc[:, 2, :])
    hists_c = _histograms([flat_c], _par(jnp.stack([mn_c]),
                                         jnp.stack([mx_c])))

    view_a = feat_a.transpose(0, 2, 3, 1).reshape(-1, feat_a.shape[1])
    view_b = feat_b.transpose(0, 2, 3, 1).reshape(-1, feat_b.shape[1])
    mn_a, mx_a, sa_a = _stats(view_a)
    mn_b, mx_b, sa_b = _stats(view_b)
    flat_a = view_a.reshape(-1)
    flat_b = view_b.reshape(-1)
    hists_ab = _histograms(
        [flat_a, flat_b],
        _par(jnp.stack([mn_a, mn_b]), jnp.stack([mx_a, mx_b])))

    ns = [flat_a.shape[0], flat_b.shape[0], flat_c.shape[0]]
    sums = jnp.stack([sa_a, sa_b, sa_c])
    return _finish(hists_c, hists_ab, sums, ns)


# speedup vs baseline: 1.0714x; 1.0714x over previous
import functools

import jax
import jax.numpy as jnp
from jax import lax
from jax.experimental import pallas as pl
from jax.experimental.pallas import tpu as pltpu
from jax.experimental.pallas import tpu_sc as plsc

BETA = 0.1
NBINS = 256
LANES = 16
NW = 32
CH = 24576
STAT_BLK = 786432


def _stats_body(x_ref, min_ref, max_ref, sab_ref):
    i = pl.program_id(0)
    x = x_ref[...]
    bmin = jnp.min(x, keepdims=True)
    bmax = jnp.max(x, keepdims=True)
    bsum = jnp.sum(jnp.abs(x), keepdims=True)

    @pl.when(i == 0)
    def _():
        min_ref[...] = bmin
        max_ref[...] = bmax
        sab_ref[...] = bsum

    @pl.when(i != 0)
    def _():
        min_ref[...] = jnp.minimum(min_ref[...], bmin)
        max_ref[...] = jnp.maximum(max_ref[...], bmax)
        sab_ref[...] = sab_ref[...] + bsum


def _stats(x2, grid=8):
    rows, width = x2.shape
    blk = rows // grid
    out = pl.pallas_call(
        _stats_body,
        grid=(grid,),
        in_specs=[pl.BlockSpec((blk, width), lambda i: (i, 0))],
        out_specs=[pl.BlockSpec((1, 1), lambda i: (0, 0))] * 3,
        out_shape=[jax.ShapeDtypeStruct((1, 1), jnp.float32)] * 3,
    )(x2)
    return out[0][0, 0], out[1][0, 0], out[2][0, 0]


def _sc_stats(flat):
    per = int(flat.shape[0]) // NW
    nch = per // CH
    assert nch % 2 == 0
    mesh = plsc.VectorSubcoreMesh(core_axis_name="c", subcore_axis_name="s")

    def body(src, out, buf, st, sem0, sem1):
        cid = lax.axis_index("c")
        sid = lax.axis_index("s")
        wid = sid * 2 + cid
        base = wid * per
        sems = (sem0, sem1)

        pltpu.async_copy(src.at[pl.ds(base, CH)], buf.at[0], sem0)
        pltpu.async_copy(src.at[pl.ds(base + CH, CH)], buf.at[1], sem1)

        big = jnp.full((LANES,), 3.4028234e38, jnp.float32)
        zero = jnp.zeros((LANES,), jnp.float32)
        init = ((big,) * 8, ((-big),) * 8, (zero,) * 8)

        def _chunks(ci, carry):
            for b in (0, 1):
                pltpu.make_async_copy(
                    src.at[pl.ds(0, CH)], buf.at[b], sems[b]).wait()

                def _vecs(i, c):
                    mns, mxs, sas = c
                    mns, mxs, sas = list(mns), list(mxs), list(sas)
                    for u in range(8):
                        v = buf[b, pl.ds((i + u) * LANES, LANES)]
                        mns[u] = jnp.minimum(mns[u], v)
                        mxs[u] = jnp.maximum(mxs[u], v)
                        sas[u] = sas[u] + jnp.abs(v)
                    return (tuple(mns), tuple(mxs), tuple(sas))

                carry = pl.loop(0, CH // LANES, step=8,
                                init_carry=carry)(_vecs)

                nxt = ci + b + 2

                @pl.when(nxt < nch)
                def _():
                    pltpu.async_copy(src.at[pl.ds(base + nxt * CH, CH)],
                                     buf.at[b], sems[b])
            return carry

        mns, mxs, sas = pl.loop(0, nch, step=2, init_carry=init)(_chunks)
        mn, mx, sa = mns[0], mxs[0], sas[0]
        for u in range(1, 8):
            mn = jnp.minimum(mn, mns[u])
            mx = jnp.maximum(mx, mxs[u])
            sa = sa + sas[u]
        st[0, :] = mn
        st[1, :] = mx
        st[2, :] = sa
        pltpu.sync_copy(st, out.at[wid])

    kfn = pl.kernel(
        body,
        out_type=jax.ShapeDtypeStruct((NW, 3, LANES), jnp.float32),
        mesh=mesh,
        scratch_types=[
            pltpu.VMEM((2, CH), jnp.float32),
            pltpu.VMEM((3, LANES), jnp.float32),
            pltpu.SemaphoreType.DMA,
            pltpu.SemaphoreType.DMA,
        ],
        compiler_params=pltpu.CompilerParams(
            use_tc_tiling_on_sc=False, needs_layout_passes=False),
    )
    return kfn(flat)


def _histograms(flat_list, par):
    nt = len(flat_list)
    per_tiles = tuple(int(f.shape[0]) // NW for f in flat_list)
    mesh = plsc.VectorSubcoreMesh(core_axis_name="c", subcore_axis_name="s")

    def body(*refs):
        srcs = refs[:nt]
        par_ref = refs[nt]
        out = refs[nt + 1]
        buf = refs[nt + 2]
        hists = refs[nt + 3:nt + 3 + nt]
        pv = refs[nt + 3 + nt]
        sem0, sem1 = refs[nt + 4 + nt:nt + 6 + nt]

        cid = lax.axis_index("c")
        sid = lax.axis_index("s")
        wid = sid * 2 + cid

        zero16 = jnp.zeros((LANES,), jnp.float32)

        @pl.loop(0, NBINS)
        def _zero(r):
            for h in hists:
                h[r, :] = zero16

        pltpu.sync_copy(par_ref, pv)

        lanes = lax.iota(jnp.int32, LANES)
        ones = jnp.ones((LANES,), jnp.float32)
        sems = (sem0, sem1)

        for t in range(nt):
            src = srcs[t]
            hist = hists[t]
            per = per_tiles[t]
            nch = per // CH
            base = wid * per
            minv = pv[t, :]
            scv = pv[nt + t, :]

            pltpu.async_copy(src.at[pl.ds(base, CH)], buf.at[0], sem0)
            pltpu.async_copy(src.at[pl.ds(base + CH, CH)], buf.at[1], sem1)

            @pl.loop(0, nch, step=2)
            def _chunks(ci):
                for b in (0, 1):
                    cur = ci + b

                    @pl.when(cur < nch)
                    def _():
                        pltpu.make_async_copy(
                            src.at[pl.ds(0, CH)], buf.at[b], sems[b]).wait()

                        @plsc.parallel_loop(0, CH // LANES, unroll=8)
                        def _vecs(i):
                            v = buf[b, pl.ds(i * LANES, LANES)]
                            xf = (v - minv) * scv
                            bi = xf.astype(jnp.int32)
                            bi = jnp.minimum(bi, NBINS - 1)
                            plsc.addupdate_scatter(hist, [bi, lanes], ones)

                        nxt = cur + 2

                        @pl.when(nxt < nch)
                        def _():
                            pltpu.async_copy(
                                src.at[pl.ds(base + nxt * CH, CH)],
                                buf.at[b], sems[b])

        for t in range(nt):
            pltpu.sync_copy(hists[t], out.at[t, wid])

    kfn = pl.kernel(
        body,
        out_type=jax.ShapeDtypeStruct((nt, NW, NBINS, LANES), jnp.float32),
        mesh=mesh,
        scratch_types=(
            [pltpu.VMEM((2, CH), jnp.float32)]
            + [pltpu.VMEM((NBINS, LANES), jnp.float32)] * nt
            + [pltpu.VMEM((2 * nt, LANES), jnp.float32),
               pltpu.SemaphoreType.DMA,
               pltpu.SemaphoreType.DMA]),
        compiler_params=pltpu.CompilerParams(
            use_tc_tiling_on_sc=False, needs_layout_passes=False),
    )
    return kfn(*flat_list, par)


def _finish_body(ns, hc_ref, hab_ref, s_ref, o_ref):
    ent = jnp.float32(0.0)
    spars = jnp.float32(0.0)
    for t in range(3):
        n = jnp.float32(ns[t])
        href, k = (hab_ref, t) if t < 2 else (hc_ref, 0)
        h = jnp.sum(href[k], axis=1, keepdims=True)
        p = h / n
        ent = ent + (-jnp.sum(p * jnp.log2(p + 1e-08)))
        spars = spars + s_ref[0, t] / n
    o_ref[...] = jnp.reshape(spars / 3.0 + BETA * (ent / 3.0), (1, 1))


def _finish(hists_c, hists_ab, sums, ns):
    hc = hists_c.transpose(0, 2, 1, 3).reshape(1, NBINS, NW * LANES)
    hab = hists_ab.transpose(0, 2, 1, 3).reshape(2, NBINS, NW * LANES)
    out = pl.pallas_call(
        functools.partial(_finish_body, ns),
        out_shape=jax.ShapeDtypeStruct((1, 1), jnp.float32),
    )(hc, hab, sums.reshape(1, 3))
    return out[0, 0]


def _par(mins, maxs):
    scale = 256.0 / (maxs - mins + 1e-08)
    k = mins.shape[0]
    return jnp.concatenate(
        [jnp.broadcast_to(mins[:, None], (k, LANES)),
         jnp.broadcast_to(scale[:, None], (k, LANES))], axis=0)


def kernel(feat_a, feat_b, feat_c):
    flat_c = feat_c.reshape(-1)
    pc = _sc_stats(flat_c)
    mn_c = jnp.min(pc[:, 0, :])
    mx_c = jnp.max(pc[:, 1, :])
    sa_c = jnp.sum(pc[:, 2, :])
    hists_c = _histograms([flat_c], _par(jnp.stack([mn_c]),
                                         jnp.stack([mx_c])))

    view_a = feat_a.transpose(0, 2, 3, 1).reshape(-1, feat_a.shape[1])
    view_b = feat_b.transpose(0, 2, 3, 1).reshape(-1, feat_b.shape[1])
    mn_a, mx_a, sa_a = _stats(view_a)
    mn_b, mx_b, sa_b = _stats(view_b)
    flat_a = view_a.reshape(-1)
    flat_b = view_b.reshape(-1)
    par_ab = (_par(jnp.stack([mn_a, mn_b]), jnp.stack([mx_a, mx_b]))
              + hists_c[0, 0, 0, 0] * 0.0)
    hists_ab = _histograms([flat_a, flat_b], par_ab)

    ns = [flat_a.shape[0], flat_b.shape[0], flat_c.shape[0]]
    sums = jnp.stack([sa_a, sa_b, sa_c])
    return _finish(hists_c, hists_ab, sums, ns)
